# baseline (speedup 1.0000x reference)
import functools

import jax
import jax.numpy as jnp
from jax import lax
from jax.experimental import pallas as pl
from jax.experimental.pallas import tpu as pltpu
from jax.experimental.pallas import tpu_sc as plsc

VOCAB = 1000
CMAIN = 896
CTAIL = VOCAB - CMAIN
NPOS = 1024 * 50
NW = 32
PER_W = NPOS // NW
CHUNK = 16
NCHUNK = PER_W // CHUNK
NBUF = 5
NGRP = NCHUNK // NBUF
LGRP = 128
NLG = PER_W // LGRP
LTAIL = PER_W - NLG * LGRP


def _lse_table_tc(table):
    def body(t_ref, o_ref):
        x = t_ref[...]
        m = jnp.max(x, axis=1)
        s = jnp.sum(jnp.exp(x - m[:, None]), axis=1)
        o_ref[...] = jnp.log(s) + m

    return pl.pallas_call(
        body,
        out_shape=jax.ShapeDtypeStruct((VOCAB,), jnp.float32),
    )(table)


def _sc_loss(table_flat, lse, idx, tgt):
    mesh = plsc.VectorSubcoreMesh(core_axis_name="c", subcore_axis_name="s")

    @functools.partial(
        pl.kernel,
        mesh=mesh,
        compiler_params=pltpu.CompilerParams(use_tc_tiling_on_sc=False),
        out_type=jax.ShapeDtypeStruct((NW * 16,), jnp.float32),
        scratch_types=[
            pltpu.VMEM((PER_W,), jnp.int32),
            pltpu.VMEM((PER_W,), jnp.int32),
            pltpu.VMEM((PER_W,), jnp.int32),
            pltpu.VMEM((PER_W,), jnp.float32),
            pltpu.VMEM((PER_W,), jnp.float32),
            pltpu.VMEM((16,), jnp.float32),
            pltpu.SemaphoreType.DMA,
            pltpu.SemaphoreType.DMA,
        ],
    )
    def k(tablef_hbm, lse_hbm, idx_hbm, tgt_hbm, part_hbm,
          idx_v, tgt_v, fidx_v, pick_v, lsev_v, acc_v, semp, seml):
        wid = lax.axis_index("s") * 2 + lax.axis_index("c")
        base0 = wid * PER_W
        pltpu.sync_copy(idx_hbm.at[pl.ds(base0, PER_W)], idx_v)
        pltpu.sync_copy(tgt_hbm.at[pl.ds(base0, PER_W)], tgt_v)

        def fbody(j, c):
            off = j * 64
            for u in range(4):
                o = pl.ds(off + u * 16, 16)
                fidx_v[o] = idx_v[o] * VOCAB + tgt_v[o]
            return c
        lax.fori_loop(0, PER_W // 64, fbody, 0)

        def lfire(off, n):
            pltpu.async_copy(tablef_hbm.at[fidx_v.at[pl.ds(off, n)]],
                             pick_v.at[pl.ds(off, n)], semp)
            pltpu.async_copy(lse_hbm.at[idx_v.at[pl.ds(off, n)]],
                             lsev_v.at[pl.ds(off, n)], seml)

        def lfireb(j, c):
            lfire(j * LGRP, LGRP)
            return c
        lax.fori_loop(0, NLG, lfireb, 0)
        lfire(NLG * LGRP, LTAIL)

        def ldrain(off, n):
            pltpu.make_async_copy(tablef_hbm.at[pl.ds(0, n)],
                                  pick_v.at[pl.ds(off, n)], semp).wait()
            pltpu.make_async_copy(lse_hbm.at[pl.ds(0, n)],
                                  lsev_v.at[pl.ds(off, n)], seml).wait()

        def ldrainb(j, c):
            ldrain(j * LGRP, LGRP)
            return c
        lax.fori_loop(0, NLG, ldrainb, 0)
        ldrain(NLG * LGRP, LTAIL)

        acc_v[...] = jnp.zeros((16,), jnp.float32)

        def abody(j, c):
            off = j * 64
            for u in range(4):
                o = pl.ds(off + u * 16, 16)
                acc_v[...] = acc_v[...] + (lsev_v[o] - pick_v[o])
            return c
        lax.fori_loop(0, PER_W // 64, abody, 0)
        pltpu.sync_copy(acc_v, part_hbm.at[pl.ds(wid * 16, 16)])

    return k(table_flat, lse, idx, tgt)


def _sc_gather(table_a, table_b, idx):
    mesh = plsc.VectorSubcoreMesh(core_axis_name="c", subcore_axis_name="s")

    @functools.partial(
        pl.kernel,
        mesh=mesh,
        out_type=[
            jax.ShapeDtypeStruct((NPOS, VOCAB), jnp.float32),
            jax.ShapeDtypeStruct((NPOS, 128), jnp.float32),
        ],
        scratch_types=[
            pltpu.VMEM((PER_W,), jnp.int32),
            pltpu.VMEM((CHUNK, CMAIN), jnp.float32),
            pltpu.VMEM((CHUNK, CMAIN), jnp.float32),
            pltpu.VMEM((CHUNK, CMAIN), jnp.float32),
            pltpu.VMEM((CHUNK, CMAIN), jnp.float32),
            pltpu.VMEM((CHUNK, CMAIN), jnp.float32),
            pltpu.VMEM((CHUNK, 128), jnp.float32),
            pltpu.VMEM((CHUNK, 128), jnp.float32),
            pltpu.VMEM((CHUNK, 128), jnp.float32),
            pltpu.VMEM((CHUNK, 128), jnp.float32),
            pltpu.VMEM((CHUNK, 128), jnp.float32),
            pltpu.SemaphoreType.DMA,
            pltpu.SemaphoreType.DMA,
            pltpu.SemaphoreType.DMA,
            pltpu.SemaphoreType.DMA,
            pltpu.SemaphoreType.DMA,
            pltpu.SemaphoreType.DMA,
            pltpu.SemaphoreType.DMA,
            pltpu.SemaphoreType.DMA,
            pltpu.SemaphoreType.DMA,
            pltpu.SemaphoreType.DMA,
        ],
    )
    def k(ta_hbm, tb_hbm, idx_hbm, out_hbm, tail_hbm,
          idx_v, ra0, ra1, ra2, ra3, ra4, rb0, rb1, rb2, rb3, rb4,
          sg0, sg1, sg2, sg3, sg4, ss0, ss1, ss2, ss3, ss4):
        abufs = [ra0, ra1, ra2, ra3, ra4]
        bbufs = [rb0, rb1, rb2, rb3, rb4]
        gsems = [sg0, sg1, sg2, sg3, sg4]
        ssems = [ss0, ss1, ss2, ss3, ss4]
        wid = lax.axis_index("s") * 2 + lax.axis_index("c")
        base0 = wid * PER_W
        pltpu.sync_copy(idx_hbm.at[pl.ds(base0, PER_W)], idx_v)

        def gstart(c, b):
            ids = idx_v.at[pl.ds(c * CHUNK, CHUNK)]
            pltpu.async_copy(ta_hbm.at[ids], abufs[b], gsems[b])
            pltpu.async_copy(tb_hbm.at[ids], bbufs[b], gsems[b])

        def gwait(b):
            pltpu.make_async_copy(ta_hbm.at[pl.ds(0, CHUNK)],
                                  abufs[b], gsems[b]).wait()
            pltpu.make_async_copy(tb_hbm.at[pl.ds(0, CHUNK)],
                                  bbufs[b], gsems[b]).wait()

        def sstart(c, b):
            rows = pl.ds(base0 + c * CHUNK, CHUNK)
            pltpu.async_copy(abufs[b],
                             out_hbm.at[rows, pl.ds(0, CMAIN)], ssems[b])
            pltpu.async_copy(bbufs[b], tail_hbm.at[rows], ssems[b])

        def swait(b):
            pltpu.make_async_copy(abufs[b],
                                  out_hbm.at[pl.ds(0, CHUNK), pl.ds(0, CMAIN)],
                                  ssems[b]).wait()
            pltpu.make_async_copy(bbufs[b],
                                  tail_hbm.at[pl.ds(0, CHUNK)], ssems[b]).wait()

        gstart(0, 0)
        gstart(1, 1)

        def obody(q, carry):
            for b in range(NBUF):
                c = q * NBUF + b
                gwait(b)
                sstart(c, b)
                nb = (b + 2) % NBUF

                @pl.when(c >= NBUF - 2)
                def _():
                    swait(nb)

                @pl.when(c + 2 < NCHUNK)
                def _():
                    gstart(c + 2, nb)
            return carry

        lax.fori_loop(0, NGRP, obody, 0)
        for t in range(NBUF - 2, 0, -1):
            swait((NCHUNK - t) % NBUF)

    return k(table_a, table_b, idx)


def _stitch_tail_tc(sc_out, tail):
    def body(o_in_ref, t_ref, o_ref):
        del o_in_ref
        o_ref[...] = t_ref[...]

    nblk = NPOS // RSTITCH
    return pl.pallas_call(
        body,
        grid=(nblk,),
        in_specs=[
            pl.BlockSpec((RSTITCH, 128), lambda i: (i, CMAIN // 128)),
            pl.BlockSpec((RSTITCH, 128), lambda i: (i, 0)),
        ],
        out_specs=pl.BlockSpec((RSTITCH, 128), lambda i: (i, CMAIN // 128)),
        out_shape=jax.ShapeDtypeStruct((NPOS, VOCAB), jnp.float32),
        input_output_aliases={0: 0},
    )(sc_out, tail)


def kernel(idxs, targets, table):
    idx = idxs.reshape(-1).astype(jnp.int32)
    tgt = targets.reshape(-1).astype(jnp.int32)
    table = table.astype(jnp.float32)
    lse = _lse_table_tc(table)
    table_flat = jnp.pad(table.reshape(-1), (0, 8))
    table_a = table[:, :CMAIN]
    table_b = jnp.pad(table[:, CMAIN:], ((0, 0), (0, 128 - CTAIL)))
    sc_out, tail = _sc_gather(table_a, table_b, idx)
    lse_dep = lse + sc_out[0, 0] * 0.0
    part = _sc_loss(table_flat, lse_dep, idx, tgt)
    out = lax.dynamic_update_slice(sc_out, tail[:, :CTAIL], (0, CMAIN))
    loss = jnp.sum(part) / NPOS
    return (out, loss)


# speedup vs baseline: 1.0915x; 1.0915x over previous
import functools

import jax
import jax.numpy as jnp
from jax import lax
from jax.experimental import pallas as pl
from jax.experimental.pallas import tpu as pltpu
from jax.experimental.pallas import tpu_sc as plsc

VOCAB = 1000
CMAIN = 896
CTAIL = VOCAB - CMAIN
NPOS = 1024 * 50
NW = 32
PER_W = NPOS // NW
CHUNK = 16
NCHUNK = PER_W // CHUNK
NBUF = 5
NGRP = NCHUNK // NBUF
LGRP = 128
NLG = PER_W // LGRP
LTAIL = PER_W - NLG * LGRP


def _lse_table_tc(table):
    def body(t_ref, o_ref):
        x = t_ref[...]
        m = jnp.max(x, axis=1)
        s = jnp.sum(jnp.exp(x - m[:, None]), axis=1)
        o_ref[...] = jnp.log(s) + m

    return pl.pallas_call(
        body,
        out_shape=jax.ShapeDtypeStruct((VOCAB,), jnp.float32),
    )(table)


def _sc_loss(table_flat, lse, idx, tgt):
    mesh = plsc.VectorSubcoreMesh(core_axis_name="c", subcore_axis_name="s")

    @functools.partial(
        pl.kernel,
        mesh=mesh,
        compiler_params=pltpu.CompilerParams(use_tc_tiling_on_sc=False),
        out_type=jax.ShapeDtypeStruct((NW * 16,), jnp.float32),
        scratch_types=[
            pltpu.VMEM((PER_W,), jnp.int32),
            pltpu.VMEM((PER_W,), jnp.int32),
            pltpu.VMEM((PER_W,), jnp.int32),
            pltpu.VMEM((PER_W,), jnp.float32),
            pltpu.VMEM((PER_W,), jnp.float32),
            pltpu.VMEM((16,), jnp.float32),
            pltpu.SemaphoreType.DMA,
            pltpu.SemaphoreType.DMA,
        ],
    )
    def k(tablef_hbm, lse_hbm, idx_hbm, tgt_hbm, part_hbm,
          idx_v, tgt_v, fidx_v, pick_v, lsev_v, acc_v, semp, seml):
        wid = lax.axis_index("s") * 2 + lax.axis_index("c")
        base0 = wid * PER_W
        pltpu.sync_copy(idx_hbm.at[pl.ds(base0, PER_W)], idx_v)
        pltpu.sync_copy(tgt_hbm.at[pl.ds(base0, PER_W)], tgt_v)

        def fbody(j, c):
            off = j * 64
            for u in range(4):
                o = pl.ds(off + u * 16, 16)
                fidx_v[o] = idx_v[o] * VOCAB + tgt_v[o]
            return c
        lax.fori_loop(0, PER_W // 64, fbody, 0)

        def lfire(off, n):
            pltpu.async_copy(tablef_hbm.at[fidx_v.at[pl.ds(off, n)]],
                             pick_v.at[pl.ds(off, n)], semp)
            pltpu.async_copy(lse_hbm.at[idx_v.at[pl.ds(off, n)]],
                             lsev_v.at[pl.ds(off, n)], seml)

        def lfireb(j, c):
            lfire(j * LGRP, LGRP)
            return c
        lax.fori_loop(0, NLG, lfireb, 0)
        lfire(NLG * LGRP, LTAIL)

        def ldrain(off, n):
            pltpu.make_async_copy(tablef_hbm.at[pl.ds(0, n)],
                                  pick_v.at[pl.ds(off, n)], semp).wait()
            pltpu.make_async_copy(lse_hbm.at[pl.ds(0, n)],
                                  lsev_v.at[pl.ds(off, n)], seml).wait()

        def ldrainb(j, c):
            ldrain(j * LGRP, LGRP)
            return c
        lax.fori_loop(0, NLG, ldrainb, 0)
        ldrain(NLG * LGRP, LTAIL)

        acc_v[...] = jnp.zeros((16,), jnp.float32)

        def abody(j, c):
            off = j * 64
            for u in range(4):
                o = pl.ds(off + u * 16, 16)
                acc_v[...] = acc_v[...] + (lsev_v[o] - pick_v[o])
            return c
        lax.fori_loop(0, PER_W // 64, abody, 0)
        pltpu.sync_copy(acc_v, part_hbm.at[pl.ds(wid * 16, 16)])

    return k(table_flat, lse, idx, tgt)


def _sc_gather(table_a, table_b, idx):
    mesh = plsc.VectorSubcoreMesh(core_axis_name="c", subcore_axis_name="s")

    @functools.partial(
        pl.kernel,
        mesh=mesh,
        out_type=[
            jax.ShapeDtypeStruct((NPOS, VOCAB), jnp.float32),
            jax.ShapeDtypeStruct((NPOS, 128), jnp.float32),
        ],
        scratch_types=[
            pltpu.VMEM((PER_W,), jnp.int32),
            pltpu.VMEM((CHUNK, CMAIN), jnp.float32),
            pltpu.VMEM((CHUNK, CMAIN), jnp.float32),
            pltpu.VMEM((CHUNK, CMAIN), jnp.float32),
            pltpu.VMEM((CHUNK, CMAIN), jnp.float32),
            pltpu.VMEM((CHUNK, CMAIN), jnp.float32),
            pltpu.VMEM((CHUNK, 128), jnp.float32),
            pltpu.VMEM((CHUNK, 128), jnp.float32),
            pltpu.VMEM((CHUNK, 128), jnp.float32),
            pltpu.VMEM((CHUNK, 128), jnp.float32),
            pltpu.VMEM((CHUNK, 128), jnp.float32),
            pltpu.SemaphoreType.DMA,
            pltpu.SemaphoreType.DMA,
            pltpu.SemaphoreType.DMA,
            pltpu.SemaphoreType.DMA,
            pltpu.SemaphoreType.DMA,
            pltpu.SemaphoreType.DMA,
            pltpu.SemaphoreType.DMA,
            pltpu.SemaphoreType.DMA,
            pltpu.SemaphoreType.DMA,
            pltpu.SemaphoreType.DMA,
        ],
    )
    def k(ta_hbm, tb_hbm, idx_hbm, out_hbm, tail_hbm,
          idx_v, ra0, ra1, ra2, ra3, ra4, rb0, rb1, rb2, rb3, rb4,
          sg0, sg1, sg2, sg3, sg4, ss0, ss1, ss2, ss3, ss4):
        abufs = [ra0, ra1, ra2, ra3, ra4]
        bbufs = [rb0, rb1, rb2, rb3, rb4]
        gsems = [sg0, sg1, sg2, sg3, sg4]
        ssems = [ss0, ss1, ss2, ss3, ss4]
        wid = lax.axis_index("s") * 2 + lax.axis_index("c")
        base0 = wid * PER_W
        pltpu.sync_copy(idx_hbm.at[pl.ds(base0, PER_W)], idx_v)

        def gstart(c, b):
            ids = idx_v.at[pl.ds(c * CHUNK, CHUNK)]
            pltpu.async_copy(ta_hbm.at[ids], abufs[b], gsems[b])
            pltpu.async_copy(tb_hbm.at[ids], bbufs[b], gsems[b])

        def gwait(b):
            pltpu.make_async_copy(ta_hbm.at[pl.ds(0, CHUNK)],
                                  abufs[b], gsems[b]).wait()
            pltpu.make_async_copy(tb_hbm.at[pl.ds(0, CHUNK)],
                                  bbufs[b], gsems[b]).wait()

        def sstart(c, b):
            rows = pl.ds(base0 + c * CHUNK, CHUNK)
            pltpu.async_copy(abufs[b],
                             out_hbm.at[rows, pl.ds(0, CMAIN)], ssems[b])
            pltpu.async_copy(bbufs[b], tail_hbm.at[rows], ssems[b])

        def swait(b):
            pltpu.make_async_copy(abufs[b],
                                  out_hbm.at[pl.ds(0, CHUNK), pl.ds(0, CMAIN)],
                                  ssems[b]).wait()
            pltpu.make_async_copy(bbufs[b],
                                  tail_hbm.at[pl.ds(0, CHUNK)], ssems[b]).wait()

        gstart(0, 0)
        gstart(1, 1)

        def obody(q, carry):
            for b in range(NBUF):
                c = q * NBUF + b
                gwait(b)
                sstart(c, b)
                nb = (b + 2) % NBUF

                @pl.when(c >= NBUF - 2)
                def _():
                    swait(nb)

                @pl.when(c + 2 < NCHUNK)
                def _():
                    gstart(c + 2, nb)
            return carry

        lax.fori_loop(0, NGRP, obody, 0)
        for t in range(NBUF - 2, 0, -1):
            swait((NCHUNK - t) % NBUF)

    return k(table_a, table_b, idx)


def _stitch_tail_tc(sc_out, tail):
    def body(o_in_ref, t_ref, o_ref):
        del o_in_ref
        o_ref[...] = t_ref[...]

    nblk = NPOS // RSTITCH
    return pl.pallas_call(
        body,
        grid=(nblk,),
        in_specs=[
            pl.BlockSpec((RSTITCH, 128), lambda i: (i, CMAIN // 128)),
            pl.BlockSpec((RSTITCH, 128), lambda i: (i, 0)),
        ],
        out_specs=pl.BlockSpec((RSTITCH, 128), lambda i: (i, CMAIN // 128)),
        out_shape=jax.ShapeDtypeStruct((NPOS, VOCAB), jnp.float32),
        input_output_aliases={0: 0},
    )(sc_out, tail)


def kernel(idxs, targets, table):
    idx = idxs.reshape(-1).astype(jnp.int32)
    tgt = targets.reshape(-1).astype(jnp.int32)
    table = table.astype(jnp.float32)
    lse = _lse_table_tc(table)
    table_flat = jnp.pad(table.reshape(-1), (0, 8))
    table_a = table[:, :CMAIN]
    table_b = jnp.pad(table[:, CMAIN:], ((0, 0), (0, 128 - CTAIL)))
    part = _sc_loss(table_flat, lse, idx, tgt)
    sc_out, tail = _sc_gather(table_a, table_b, idx)
    out = lax.dynamic_update_slice(sc_out, tail[:, :CTAIL], (0, CMAIN))
    loss = jnp.sum(part) / NPOS
    return (out, loss)


# speedup vs baseline: 1.0915x; 1.0000x over previous
import functools

import jax
import jax.numpy as jnp
from jax import lax
from jax.experimental import pallas as pl
from jax.experimental.pallas import tpu as pltpu
from jax.experimental.pallas import tpu_sc as plsc

VOCAB = 1000
CMAIN = 896
CTAIL = VOCAB - CMAIN
NPOS = 1024 * 50
NW = 32
PER_W = NPOS // NW
CHUNK = 16
NCHUNK = PER_W // CHUNK
NBUF = 5
NGRP = NCHUNK // NBUF
LGRP = 128
NLG = PER_W // LGRP
LTAIL = PER_W - NLG * LGRP


def _lse_table_tc(table):
    def body(t_ref, o_ref):
        x = t_ref[...]
        m = jnp.max(x, axis=1)
        s = jnp.sum(jnp.exp(x - m[:, None]), axis=1)
        o_ref[...] = jnp.log(s) + m

    return pl.pallas_call(
        body,
        out_shape=jax.ShapeDtypeStruct((VOCAB,), jnp.float32),
    )(table)


def _sc_loss(table_flat, lse, idx, tgt):
    mesh = plsc.VectorSubcoreMesh(core_axis_name="c", subcore_axis_name="s")

    @functools.partial(
        pl.kernel,
        mesh=mesh,
        compiler_params=pltpu.CompilerParams(use_tc_tiling_on_sc=False),
        out_type=jax.ShapeDtypeStruct((NW * 16,), jnp.float32),
        scratch_types=[
            pltpu.VMEM((PER_W,), jnp.int32),
            pltpu.VMEM((PER_W,), jnp.int32),
            pltpu.VMEM((PER_W,), jnp.int32),
            pltpu.VMEM((PER_W,), jnp.float32),
            pltpu.VMEM((PER_W,), jnp.float32),
            pltpu.VMEM((16,), jnp.float32),
            pltpu.SemaphoreType.DMA,
            pltpu.SemaphoreType.DMA,
        ],
    )
    def k(tablef_hbm, lse_hbm, idx_hbm, tgt_hbm, part_hbm,
          idx_v, tgt_v, fidx_v, pick_v, lsev_v, acc_v, semp, seml):
        wid = lax.axis_index("s") * 2 + lax.axis_index("c")
        base0 = wid * PER_W
        pltpu.sync_copy(idx_hbm.at[pl.ds(base0, PER_W)], idx_v)
        pltpu.sync_copy(tgt_hbm.at[pl.ds(base0, PER_W)], tgt_v)

        def fbody(j, c):
            off = j * 64
            for u in range(4):
                o = pl.ds(off + u * 16, 16)
                fidx_v[o] = idx_v[o] * VOCAB + tgt_v[o]
            return c
        lax.fori_loop(0, PER_W // 64, fbody, 0)

        def lfire(off, n):
            pltpu.async_copy(tablef_hbm.at[fidx_v.at[pl.ds(off, n)]],
                             pick_v.at[pl.ds(off, n)], semp)
            pltpu.async_copy(lse_hbm.at[idx_v.at[pl.ds(off, n)]],
                             lsev_v.at[pl.ds(off, n)], seml)

        def lfireb(j, c):
            lfire(j * LGRP, LGRP)
            return c
        lax.fori_loop(0, NLG, lfireb, 0)
        lfire(NLG * LGRP, LTAIL)

        def ldrain(off, n):
            pltpu.make_async_copy(tablef_hbm.at[pl.ds(0, n)],
                                  pick_v.at[pl.ds(off, n)], semp).wait()
            pltpu.make_async_copy(lse_hbm.at[pl.ds(0, n)],
                                  lsev_v.at[pl.ds(off, n)], seml).wait()

        def ldrainb(j, c):
            ldrain(j * LGRP, LGRP)
            return c
        lax.fori_loop(0, NLG, ldrainb, 0)
        ldrain(NLG * LGRP, LTAIL)

        acc_v[...] = jnp.zeros((16,), jnp.float32)

        def abody(j, c):
            off = j * 64
            for u in range(4):
                o = pl.ds(off + u * 16, 16)
                acc_v[...] = acc_v[...] + (lsev_v[o] - pick_v[o])
            return c
        lax.fori_loop(0, PER_W // 64, abody, 0)
        pltpu.sync_copy(acc_v, part_hbm.at[pl.ds(wid * 16, 16)])

    return k(table_flat, lse, idx, tgt)


def _sc_gather(table_a, table_b, idx):
    mesh = plsc.VectorSubcoreMesh(core_axis_name="c", subcore_axis_name="s")

    @functools.partial(
        pl.kernel,
        mesh=mesh,
        out_type=[
            jax.ShapeDtypeStruct((NPOS, VOCAB), jnp.float32),
            jax.ShapeDtypeStruct((NPOS, 128), jnp.float32),
        ],
        scratch_types=[
            pltpu.VMEM((PER_W,), jnp.int32),
            pltpu.VMEM((CHUNK, CMAIN), jnp.float32),
            pltpu.VMEM((CHUNK, CMAIN), jnp.float32),
            pltpu.VMEM((CHUNK, CMAIN), jnp.float32),
            pltpu.VMEM((CHUNK, CMAIN), jnp.float32),
            pltpu.VMEM((CHUNK, CMAIN), jnp.float32),
            pltpu.VMEM((CHUNK, 128), jnp.float32),
            pltpu.VMEM((CHUNK, 128), jnp.float32),
            pltpu.VMEM((CHUNK, 128), jnp.float32),
            pltpu.VMEM((CHUNK, 128), jnp.float32),
            pltpu.VMEM((CHUNK, 128), jnp.float32),
            pltpu.SemaphoreType.DMA,
            pltpu.SemaphoreType.DMA,
            pltpu.SemaphoreType.DMA,
            pltpu.SemaphoreType.DMA,
            pltpu.SemaphoreType.DMA,
            pltpu.SemaphoreType.DMA,
            pltpu.SemaphoreType.DMA,
            pltpu.SemaphoreType.DMA,
            pltpu.SemaphoreType.DMA,
            pltpu.SemaphoreType.DMA,
        ],
    )
    def k(ta_hbm, tb_hbm, idx_hbm, out_hbm, tail_hbm,
          idx_v, ra0, ra1, ra2, ra3, ra4, rb0, rb1, rb2, rb3, rb4,
          sg0, sg1, sg2, sg3, sg4, ss0, ss1, ss2, ss3, ss4):
        abufs = [ra0, ra1, ra2, ra3, ra4]
        bbufs = [rb0, rb1, rb2, rb3, rb4]
        gsems = [sg0, sg1, sg2, sg3, sg4]
        ssems = [ss0, ss1, ss2, ss3, ss4]
        wid = lax.axis_index("s") * 2 + lax.axis_index("c")
        base0 = wid * PER_W
        pltpu.sync_copy(idx_hbm.at[pl.ds(base0, PER_W)], idx_v)

        def gstart(c, b):
            ids = idx_v.at[pl.ds(c * CHUNK, CHUNK)]
            pltpu.async_copy(ta_hbm.at[ids], abufs[b], gsems[b])
            pltpu.async_copy(tb_hbm.at[ids], bbufs[b], gsems[b])

        def gwait(b):
            pltpu.make_async_copy(ta_hbm.at[pl.ds(0, CHUNK)],
                                  abufs[b], gsems[b]).wait()
            pltpu.make_async_copy(tb_hbm.at[pl.ds(0, CHUNK)],
                                  bbufs[b], gsems[b]).wait()

        def sstart(c, b):
            rows = pl.ds(base0 + c * CHUNK, CHUNK)
            pltpu.async_copy(abufs[b],
                             out_hbm.at[rows, pl.ds(0, CMAIN)], ssems[b])
            pltpu.async_copy(bbufs[b], tail_hbm.at[rows], ssems[b])

        def swait(b):
            pltpu.make_async_copy(abufs[b],
                                  out_hbm.at[pl.ds(0, CHUNK), pl.ds(0, CMAIN)],
                                  ssems[b]).wait()
            pltpu.make_async_copy(bbufs[b],
                                  tail_hbm.at[pl.ds(0, CHUNK)], ssems[b]).wait()

        gstart(0, 0)
        gstart(1, 1)

        def obody(q, carry):
            for b in range(NBUF):
                c = q * NBUF + b
                gwait(b)
                sstart(c, b)
                nb = (b + 2) % NBUF

                @pl.when(c >= NBUF - 2)
                def _():
                    swait(nb)

                @pl.when(c + 2 < NCHUNK)
                def _():
                    gstart(c + 2, nb)
            return carry

        lax.fori_loop(0, NGRP, obody, 0)
        for t in range(NBUF - 2, 0, -1):
            swait((NCHUNK - t) % NBUF)

    return k(table_a, table_b, idx)


def kernel(idxs, targets, table):
    idx = idxs.reshape(-1).astype(jnp.int32)
    tgt = targets.reshape(-1).astype(jnp.int32)
    table = table.astype(jnp.float32)
    lse = _lse_table_tc(table)
    table_flat = jnp.pad(table.reshape(-1), (0, 8))
    table_a = table[:, :CMAIN]
    table_b = jnp.pad(table[:, CMAIN:], ((0, 0), (0, 128 - CTAIL)))
    part = _sc_loss(table_flat, lse, idx, tgt)
    sc_out, tail = _sc_gather(table_a, table_b, idx)
    out = lax.dynamic_update_slice(sc_out, tail[:, :CTAIL], (0, CMAIN))
    loss = jnp.sum(part) / NPOS
    return (out, loss)
